# baseline (speedup 1.0000x reference)
import jax
import jax.numpy as jnp
from jax.experimental import pallas as pl
from jax.experimental.pallas import tpu as pltpu

B, C, H, Q, D = 32, 2048, 16, 4, 64
TOPK = 8
BPB = 2
NRING = 2
NSTEP = B // BPB


def _fused_kernel(wq_ref, kf_hbm, idx_ref, val_ref, scores_ref,
                  b0, b1, s0, s1):
    bufs = (b0, b1)
    sems = (s0, s1)

    def dma(step, j):
        return pltpu.make_async_copy(
            kf_hbm.at[pl.ds(step * BPB, BPB)], bufs[j], sems[j])

    for j in range(NRING):
        dma(j, j).start()

    def body(g, carry):
        for j in range(NRING):
            step = g * NRING + j
            dma(step, j).wait()
            for nb in range(BPB):
                b = step * BPB + nb
                w = wq_ref[b]
                s = jax.lax.dot_general(
                    w, bufs[j][nb], (((1,), (1,)), ((), ())),
                    preferred_element_type=jnp.float32)
                scores_ref[pl.ds(b, 1), :] = jnp.max(s, axis=0, keepdims=True)

            @pl.when(step + NRING < NSTEP)
            def _():
                dma(step + NRING, j).start()
        return carry

    jax.lax.fori_loop(0, NSTEP // NRING, body, 0)

    s = scores_ref[...]
    iota = jax.lax.broadcasted_iota(jnp.int32, (B, C), 1)
    iota8 = jax.lax.broadcasted_iota(jnp.int32, (B, TOPK), 1)
    idxs = jnp.zeros((B, TOPK), jnp.int32)
    vals = jnp.zeros((B, TOPK), jnp.float32)
    for i in range(TOPK):
        rm = jnp.max(s, axis=1, keepdims=True)
        ridx = jnp.min(jnp.where(s == rm, iota, C), axis=1, keepdims=True)
        vals = jnp.where(iota8 == i, rm, vals)
        idxs = jnp.where(iota8 == i, ridx, idxs)
        s = jnp.where(iota == ridx, jnp.float32(-jnp.inf), s)
    idx_ref[...] = idxs
    val_ref[...] = vals


@jax.jit
def _run(queries, pooled_keys):
    q = queries * jnp.float32(0.125)
    eye = jnp.eye(H, dtype=jnp.float32)
    wqt = jnp.einsum('bhqd,hg->bhqgd', q, eye).reshape(B, H * Q, H * D)
    kf = pooled_keys.reshape(B, C, H * D)
    idx, val = pl.pallas_call(
        _fused_kernel,
        in_specs=[
            pl.BlockSpec(memory_space=pltpu.MemorySpace.VMEM),
            pl.BlockSpec(memory_space=pl.ANY),
        ],
        out_specs=[
            pl.BlockSpec(memory_space=pltpu.MemorySpace.VMEM),
            pl.BlockSpec(memory_space=pltpu.MemorySpace.VMEM),
        ],
        out_shape=[
            jax.ShapeDtypeStruct((B, TOPK), jnp.int32),
            jax.ShapeDtypeStruct((B, TOPK), jnp.float32),
        ],
        scratch_shapes=(
            [pltpu.VMEM((B, C), jnp.float32)]
            + [pltpu.VMEM((BPB, C, H * D), jnp.float32) for _ in range(NRING)]
            + [pltpu.SemaphoreType.DMA for _ in range(NRING)]
        ),
    )(wqt, kf)
    return idx, val


def kernel(queries, pooled_keys):
    return _run(queries, pooled_keys)
